# Initial kernel scaffold; baseline (speedup 1.0000x reference)
import functools

import jax
import jax.numpy as jnp
from jax import lax
from jax.experimental import pallas as pl
from jax.experimental.pallas import tpu as pltpu
from jax.experimental.pallas import tpu_sc as plsc

N_NODES = 10000
D = 128
NC = 2
NS = 16
NW = NC * NS
CHUNK = 128
CPT = 80
EPT = CHUNK * CPT
E_PAD = NW * EPT
NPAD = 10240
RPT = NPAD // NS


def _sc_segment_sum(x, srcs, dsts, zeros):
    mesh = plsc.VectorSubcoreMesh(core_axis_name="c", subcore_axis_name="s")

    @functools.partial(
        pl.kernel,
        mesh=mesh,
        out_type=jax.ShapeDtypeStruct((NC, NPAD, D), jnp.float32),
        scratch_types=[
            pltpu.VMEM((CPT, CHUNK), jnp.int32),
            pltpu.VMEM((CPT, CHUNK), jnp.int32),
            pltpu.VMEM((CHUNK, D), jnp.float32),
            pltpu.VMEM((CHUNK, D), jnp.float32),
            pltpu.VMEM_SHARED((NPAD, D), jnp.float32),
            pltpu.SemaphoreType.DMA,
            pltpu.SemaphoreType.DMA,
        ],
    )
    def k(x_hbm, src_hbm, dst_hbm, zero_hbm, out_hbm,
          src_v, dst_v, buf_a, buf_b, acc, sem_a, sem_b):
        c = lax.axis_index("c")
        s = lax.axis_index("s")
        w = s * NC + c
        pltpu.sync_copy(src_hbm.at[w], src_v)
        pltpu.sync_copy(dst_hbm.at[w], dst_v)
        pltpu.sync_copy(zero_hbm.at[pl.ds(s * RPT, RPT)],
                        acc.at[pl.ds(s * RPT, RPT)])
        plsc.subcore_barrier()

        pltpu.async_copy(x_hbm.at[src_v.at[0]], buf_a, sem_a)

        def step(g, carry):
            c0 = 2 * g
            pltpu.async_copy(x_hbm.at[src_v.at[c0 + 1]], buf_b, sem_b)
            pltpu.make_async_copy(x_hbm.at[src_v.at[c0]], buf_a, sem_a).wait()
            pltpu.sync_copy(buf_a, acc.at[dst_v.at[c0]], add=True)

            @pl.when(g + 1 < CPT // 2)
            def _():
                pltpu.async_copy(x_hbm.at[src_v.at[c0 + 2]], buf_a, sem_a)

            pltpu.make_async_copy(
                x_hbm.at[src_v.at[c0 + 1]], buf_b, sem_b).wait()
            pltpu.sync_copy(buf_b, acc.at[dst_v.at[c0 + 1]], add=True)
            return carry

        lax.fori_loop(0, CPT // 2, step, 0)
        plsc.subcore_barrier()
        pltpu.sync_copy(acc.at[pl.ds(s * RPT, RPT)],
                        out_hbm.at[c, pl.ds(s * RPT, RPT)])

    return k(x, srcs, dsts, zeros)


def _tc_finish(parts, w):
    blk = 1000

    def body(p_ref, w_ref, o_ref):
        a = p_ref[0] + p_ref[1]
        h = lax.dot_general(a, w_ref[...], (((1,), (1,)), ((), ())),
                            preferred_element_type=jnp.float32,
                            precision=lax.Precision.HIGHEST)
        o_ref[...] = jnp.maximum(h, 0.0)

    return pl.pallas_call(
        body,
        grid=(N_NODES // blk,),
        in_specs=[
            pl.BlockSpec((NC, blk, D), lambda i: (0, i, 0)),
            pl.BlockSpec((D, D), lambda i: (0, 0)),
        ],
        out_specs=pl.BlockSpec((blk, D), lambda i: (i, 0)),
        out_shape=jax.ShapeDtypeStruct((N_NODES, D), jnp.float32),
    )(parts, w)


def kernel(x, edge_index, W):
    src = edge_index[0]
    dst = edge_index[1]
    e = src.shape[0]
    pad = E_PAD - e
    pad_src = (jnp.arange(pad, dtype=jnp.int32) * 131) % N_NODES
    pad_dst = N_NODES + (jnp.arange(pad, dtype=jnp.int32) % (NPAD - N_NODES))
    src_p = jnp.concatenate([src, pad_src]).reshape(NW, CPT, CHUNK)
    dst_p = jnp.concatenate([dst, pad_dst]).reshape(NW, CPT, CHUNK)
    zeros = jnp.zeros((NPAD, D), jnp.float32)
    parts = _sc_segment_sum(x, src_p, dst_p, zeros)
    return _tc_finish(parts, W)


# speedup vs baseline: 10.7247x; 10.7247x over previous
import functools

import jax
import jax.numpy as jnp
from jax import lax
from jax.experimental import pallas as pl
from jax.experimental.pallas import tpu as pltpu
from jax.experimental.pallas import tpu_sc as plsc

N_NODES = 10000
D = 128
DH = D // 2
NC = 2
NS = 16
CHUNK = 128
NB = 4
CPT = 160
EPT = CHUNK * CPT
E_PAD = NS * EPT
NPAD = 10240
RPT = NPAD // NS


def _sc_segment_sum(xf, srcs, dsts, zeros):
    mesh = plsc.VectorSubcoreMesh(core_axis_name="c", subcore_axis_name="s")

    @functools.partial(
        pl.kernel,
        mesh=mesh,
        compiler_params=pltpu.CompilerParams(use_tc_tiling_on_sc=False),
        out_type=jax.ShapeDtypeStruct((NC, NPAD, DH), jnp.float32),
        scratch_types=[
            pltpu.VMEM((CPT, CHUNK), jnp.int32),
            pltpu.VMEM((CPT, CHUNK), jnp.int32),
            [pltpu.VMEM((CHUNK, DH), jnp.float32)] * NB,
            pltpu.VMEM_SHARED((NPAD, DH), jnp.float32),
            [pltpu.SemaphoreType.DMA] * NB,
            [pltpu.SemaphoreType.DMA] * NB,
        ],
    )
    def k(x_hbm, src_hbm, dst_hbm, zero_hbm, out_hbm,
          src_v, dst_v, bufs, acc, gsem, ssem):
        c = lax.axis_index("c")
        s = lax.axis_index("s")
        pltpu.sync_copy(src_hbm.at[c, s], src_v)
        pltpu.sync_copy(dst_hbm.at[s], dst_v)
        pltpu.sync_copy(zero_hbm.at[pl.ds(s * RPT, RPT)],
                        acc.at[pl.ds(s * RPT, RPT)])
        plsc.subcore_barrier()

        for b in range(NB):
            pltpu.async_copy(x_hbm.at[src_v.at[b]], bufs[b], gsem[b])

        def step(g, carry):
            base = NB * g
            for b in range(NB):
                pltpu.make_async_copy(
                    x_hbm.at[src_v.at[base + b]], bufs[b], gsem[b]).wait()
                pltpu.async_copy(
                    bufs[b], acc.at[dst_v.at[base + b]], ssem[b], add=True)
            for b in range(NB):
                @pl.when(base + b + NB < CPT)
                def _(b=b):
                    pltpu.make_async_copy(
                        bufs[b], acc.at[dst_v.at[base + b]], ssem[b]).wait()
                    pltpu.async_copy(
                        x_hbm.at[src_v.at[base + b + NB]], bufs[b], gsem[b])
            return carry

        lax.fori_loop(0, CPT // NB, step, 0)
        for b in range(NB):
            pltpu.make_async_copy(
                bufs[b], acc.at[dst_v.at[CPT - NB + b]], ssem[b]).wait()
        plsc.subcore_barrier()
        pltpu.sync_copy(acc.at[pl.ds(s * RPT, RPT)],
                        out_hbm.at[c, pl.ds(s * RPT, RPT)])

    return k(xf, srcs, dsts, zeros)


def _tc_finish(parts, w):
    blk = 1000

    def body(p_ref, w_ref, o_ref):
        a = jnp.concatenate([p_ref[0], p_ref[1]], axis=1)
        h = lax.dot_general(a, w_ref[...], (((1,), (1,)), ((), ())),
                            preferred_element_type=jnp.float32,
                            precision=lax.Precision.HIGHEST)
        o_ref[...] = jnp.maximum(h, 0.0)

    return pl.pallas_call(
        body,
        grid=(N_NODES // blk,),
        in_specs=[
            pl.BlockSpec((NC, blk, DH), lambda i: (0, i, 0)),
            pl.BlockSpec((D, D), lambda i: (0, 0)),
        ],
        out_specs=pl.BlockSpec((blk, D), lambda i: (i, 0)),
        out_shape=jax.ShapeDtypeStruct((N_NODES, D), jnp.float32),
    )(parts, w)


def kernel(x, edge_index, W):
    src = edge_index[0]
    dst = edge_index[1]
    e = src.shape[0]
    pad = E_PAD - e
    pad_src = (jnp.arange(pad, dtype=jnp.int32) * 131) % N_NODES
    pad_dst = N_NODES + (jnp.arange(pad, dtype=jnp.int32) % (NPAD - N_NODES))
    x2 = x.reshape(2 * N_NODES, DH)
    src_p = (2 * jnp.concatenate([src, pad_src])).reshape(NS, CPT, CHUNK)
    srcs = jnp.stack([src_p, src_p + 1])
    dsts = jnp.concatenate([dst, pad_dst]).reshape(NS, CPT, CHUNK)
    zeros = jnp.zeros((NPAD, DH), jnp.float32)
    parts = _sc_segment_sum(x2, srcs, dsts, zeros)
    return _tc_finish(parts, W)


# speedup vs baseline: 11.2215x; 1.0463x over previous
import functools

import jax
import jax.numpy as jnp
from jax import lax
from jax.experimental import pallas as pl
from jax.experimental.pallas import tpu as pltpu
from jax.experimental.pallas import tpu_sc as plsc

N_NODES = 10000
D = 128
DH = D // 2
NC = 2
NS = 16
CHUNK = 128
NB = 4
CPT = 160
EPT = CHUNK * CPT
E_PAD = NS * EPT
NPAD = 10240
RPT = NPAD // NS


def _sc_segment_sum(xf, srcs, dsts):
    mesh = plsc.VectorSubcoreMesh(core_axis_name="c", subcore_axis_name="s")

    @functools.partial(
        pl.kernel,
        mesh=mesh,
        compiler_params=pltpu.CompilerParams(use_tc_tiling_on_sc=False),
        out_type=jax.ShapeDtypeStruct((NC, NPAD, DH), jnp.float32),
        scratch_types=[
            pltpu.VMEM((CPT * CHUNK,), jnp.int32),
            pltpu.VMEM((CPT, CHUNK), jnp.int32),
            [pltpu.VMEM((CHUNK, DH), jnp.float32)] * NB,
            pltpu.VMEM((CHUNK, DH), jnp.float32),
            pltpu.VMEM_SHARED((NPAD, DH), jnp.float32),
            [pltpu.SemaphoreType.DMA] * NB,
            [pltpu.SemaphoreType.DMA] * NB,
        ],
    )
    def k(x_hbm, src_hbm, dst_hbm, out_hbm,
          src_v, dst_v, bufs, zbuf, acc, gsem, ssem):
        c = lax.axis_index("c")
        s = lax.axis_index("s")
        pltpu.sync_copy(src_hbm.at[s], src_v)
        pltpu.sync_copy(dst_hbm.at[s], dst_v)

        def prep(cc):
            base = cc * CHUNK
            for j in range(CHUNK // 16):
                sl = pl.ds(base + j * 16, 16)
                src_v[sl] = src_v[sl] * 2 + c
        zv = jnp.zeros((16,), jnp.float32)
        for r in range(CHUNK):
            for j in range(DH // 16):
                zbuf[r, pl.ds(j * 16, 16)] = zv
        for t in range(RPT // CHUNK):
            pltpu.sync_copy(zbuf, acc.at[pl.ds(s * RPT + t * CHUNK, CHUNK)])
        plsc.subcore_barrier()

        for b in range(NB):
            prep(b)
            pltpu.async_copy(
                x_hbm.at[src_v.at[pl.ds(b * CHUNK, CHUNK)]], bufs[b], gsem[b])

        def step(g, carry):
            base = NB * g
            for b in range(NB):
                pltpu.make_async_copy(
                    x_hbm.at[src_v.at[pl.ds((base + b) * CHUNK, CHUNK)]],
                    bufs[b], gsem[b]).wait()
                pltpu.async_copy(
                    bufs[b], acc.at[dst_v.at[base + b]], ssem[b], add=True)
            for b in range(NB):
                @pl.when(base + b + NB < CPT)
                def _(b=b):
                    pltpu.make_async_copy(
                        bufs[b], acc.at[dst_v.at[base + b]], ssem[b]).wait()
                    prep(base + b + NB)
                    pltpu.async_copy(
                        x_hbm.at[src_v.at[pl.ds((base + b + NB) * CHUNK,
                                                CHUNK)]],
                        bufs[b], gsem[b])
            return carry

        lax.fori_loop(0, CPT // NB, step, 0)
        for b in range(NB):
            pltpu.make_async_copy(
                bufs[b], acc.at[dst_v.at[CPT - NB + b]], ssem[b]).wait()
        plsc.subcore_barrier()
        pltpu.sync_copy(acc.at[pl.ds(s * RPT, RPT)],
                        out_hbm.at[c, pl.ds(s * RPT, RPT)])

    return k(xf, srcs, dsts)


def _tc_finish(parts, w):
    blk = 2000

    def body(p_ref, w_ref, o_ref):
        a = jnp.concatenate([p_ref[0], p_ref[1]], axis=1)
        h = lax.dot_general(a, w_ref[...], (((1,), (1,)), ((), ())),
                            preferred_element_type=jnp.float32,
                            precision=lax.Precision.HIGHEST)
        o_ref[...] = jnp.maximum(h, 0.0)

    return pl.pallas_call(
        body,
        grid=(N_NODES // blk,),
        in_specs=[
            pl.BlockSpec((NC, blk, DH), lambda i: (0, i, 0)),
            pl.BlockSpec((D, D), lambda i: (0, 0)),
        ],
        out_specs=pl.BlockSpec((blk, D), lambda i: (i, 0)),
        out_shape=jax.ShapeDtypeStruct((N_NODES, D), jnp.float32),
    )(parts, w)


def kernel(x, edge_index, W):
    src = edge_index[0]
    dst = edge_index[1]
    e = src.shape[0]
    pad = E_PAD - e
    pad_src = (jnp.arange(pad, dtype=jnp.int32) * 131) % N_NODES
    pad_dst = N_NODES + (jnp.arange(pad, dtype=jnp.int32) % (NPAD - N_NODES))
    x2 = x.reshape(2 * N_NODES, DH)
    srcs = jnp.concatenate([src, pad_src]).reshape(NS, CPT * CHUNK)
    dsts = jnp.concatenate([dst, pad_dst]).reshape(NS, CPT, CHUNK)
    parts = _sc_segment_sum(x2, srcs, dsts)
    return _tc_finish(parts, W)


# speedup vs baseline: 11.5473x; 1.0290x over previous
import functools

import jax
import jax.numpy as jnp
from jax import lax
from jax.experimental import pallas as pl
from jax.experimental.pallas import tpu as pltpu
from jax.experimental.pallas import tpu_sc as plsc

N_NODES = 10000
D = 128
DH = D // 2
NC = 2
NS = 16
CHUNK = 64
NB = 8
CPT = 320
EPT = CHUNK * CPT
E_PAD = NS * EPT
NPAD = 10240
RPT = NPAD // NS


def _sc_segment_sum(xf, srcs, dsts):
    mesh = plsc.VectorSubcoreMesh(core_axis_name="c", subcore_axis_name="s")

    @functools.partial(
        pl.kernel,
        mesh=mesh,
        compiler_params=pltpu.CompilerParams(use_tc_tiling_on_sc=False),
        out_type=jax.ShapeDtypeStruct((NC, NPAD, DH), jnp.float32),
        scratch_types=[
            pltpu.VMEM((CPT * CHUNK,), jnp.int32),
            pltpu.VMEM((CPT, CHUNK), jnp.int32),
            [pltpu.VMEM((CHUNK, DH), jnp.float32)] * NB,
            pltpu.VMEM((CHUNK, DH), jnp.float32),
            pltpu.VMEM_SHARED((NPAD, DH), jnp.float32),
            [pltpu.SemaphoreType.DMA] * NB,
            [pltpu.SemaphoreType.DMA] * NB,
        ],
    )
    def k(x_hbm, src_hbm, dst_hbm, out_hbm,
          src_v, dst_v, bufs, zbuf, acc, gsem, ssem):
        c = lax.axis_index("c")
        s = lax.axis_index("s")
        pltpu.sync_copy(src_hbm.at[s], src_v)
        pltpu.sync_copy(dst_hbm.at[s], dst_v)

        def prep(cc):
            base = cc * CHUNK
            for j in range(CHUNK // 16):
                sl = pl.ds(base + j * 16, 16)
                src_v[sl] = src_v[sl] * 2 + c
        zv = jnp.zeros((16,), jnp.float32)
        for r in range(CHUNK):
            for j in range(DH // 16):
                zbuf[r, pl.ds(j * 16, 16)] = zv
        for t in range(RPT // CHUNK):
            pltpu.sync_copy(zbuf, acc.at[pl.ds(s * RPT + t * CHUNK, CHUNK)])
        plsc.subcore_barrier()

        for b in range(NB):
            prep(b)
            pltpu.async_copy(
                x_hbm.at[src_v.at[pl.ds(b * CHUNK, CHUNK)]], bufs[b], gsem[b])

        def step(g, carry):
            base = NB * g
            for b in range(NB):
                pltpu.make_async_copy(
                    x_hbm.at[src_v.at[pl.ds((base + b) * CHUNK, CHUNK)]],
                    bufs[b], gsem[b]).wait()
                pltpu.async_copy(
                    bufs[b], acc.at[dst_v.at[base + b]], ssem[b], add=True)
            for b in range(NB):
                @pl.when(base + b + NB < CPT)
                def _(b=b):
                    pltpu.make_async_copy(
                        bufs[b], acc.at[dst_v.at[base + b]], ssem[b]).wait()
                    prep(base + b + NB)
                    pltpu.async_copy(
                        x_hbm.at[src_v.at[pl.ds((base + b + NB) * CHUNK,
                                                CHUNK)]],
                        bufs[b], gsem[b])
            return carry

        lax.fori_loop(0, CPT // NB, step, 0)
        for b in range(NB):
            pltpu.make_async_copy(
                bufs[b], acc.at[dst_v.at[CPT - NB + b]], ssem[b]).wait()
        plsc.subcore_barrier()
        pltpu.sync_copy(acc.at[pl.ds(s * RPT, RPT)],
                        out_hbm.at[c, pl.ds(s * RPT, RPT)])

    return k(xf, srcs, dsts)


def _tc_finish(parts, w):
    blk = 2000

    def body(p_ref, w_ref, o_ref):
        a = jnp.concatenate([p_ref[0], p_ref[1]], axis=1)
        h = lax.dot_general(a, w_ref[...], (((1,), (1,)), ((), ())),
                            preferred_element_type=jnp.float32,
                            precision=lax.Precision.HIGHEST)
        o_ref[...] = jnp.maximum(h, 0.0)

    return pl.pallas_call(
        body,
        grid=(N_NODES // blk,),
        in_specs=[
            pl.BlockSpec((NC, blk, DH), lambda i: (0, i, 0)),
            pl.BlockSpec((D, D), lambda i: (0, 0)),
        ],
        out_specs=pl.BlockSpec((blk, D), lambda i: (i, 0)),
        out_shape=jax.ShapeDtypeStruct((N_NODES, D), jnp.float32),
    )(parts, w)


def kernel(x, edge_index, W):
    src = edge_index[0]
    dst = edge_index[1]
    e = src.shape[0]
    pad = E_PAD - e
    pad_src = (jnp.arange(pad, dtype=jnp.int32) * 131) % N_NODES
    pad_dst = N_NODES + (jnp.arange(pad, dtype=jnp.int32) % (NPAD - N_NODES))
    x2 = x.reshape(2 * N_NODES, DH)
    srcs = jnp.concatenate([src, pad_src]).reshape(NS, CPT * CHUNK)
    dsts = jnp.concatenate([dst, pad_dst]).reshape(NS, CPT, CHUNK)
    parts = _sc_segment_sum(x2, srcs, dsts)
    return _tc_finish(parts, W)
